# Initial kernel scaffold; baseline (speedup 1.0000x reference)
import jax
import jax.numpy as jnp
from jax import lax
from jax.experimental import pallas as pl

_CONF_THRES = 0.25
_IOU_THRES = 0.45
_MAX_DET = 1000
_MAX_WH = 4096.0

_N = 20000
_NPAD = 20480
_ROWS = 160
_COLS = 128
_NCLS = 80
_NIMG = 4

_NEG_INF = jnp.float32(-jnp.inf)


def _nms_body(p_ref, out_ref):
    li = (lax.broadcasted_iota(jnp.int32, (_ROWS, _COLS), 0) * _COLS
          + lax.broadcasted_iota(jnp.int32, (_ROWS, _COLS), 1))

    per_img = []
    for b in range(_NIMG):
        cx = p_ref[b, 0]
        cy = p_ref[b, 1]
        w = p_ref[b, 2]
        h = p_ref[b, 3]
        obj = p_ref[b, 4]
        x1 = cx - w / 2
        y1 = cy - h / 2
        x2 = cx + w / 2
        y2 = cy + h / 2
        best = p_ref[b, 5] * obj
        jbest = jnp.zeros((_ROWS, _COLS), jnp.int32)
        for c in range(1, _NCLS):
            v = p_ref[b, 5 + c] * obj
            take = v > best
            jbest = jnp.where(take, c, jbest)
            best = jnp.maximum(best, v)
        conf = best
        valid = (obj > _CONF_THRES) & (conf > _CONF_THRES)
        scores = jnp.where(valid, conf, _NEG_INF)
        offs = jbest.astype(jnp.float32) * _MAX_WH
        bx1 = x1 + offs
        by1 = y1 + offs
        bx2 = x2 + offs
        by2 = y2 + offs
        a2 = (bx2 - bx1) * (by2 - by1)
        jf = jbest.astype(jnp.float32)
        per_img.append((scores, x1, y1, x2, y2, bx1, by1, bx2, by2, a2, jf))

    def step(t, ss):
        new_ss = []
        for b in range(_NIMG):
            (_, x1, y1, x2, y2, bx1, by1, bx2, by2, a2, jf) = per_img[b]
            s = ss[b]
            m = jnp.max(s)
            ok = m > 0.0
            eq = s == m
            idx = jnp.min(jnp.where(eq, li, jnp.int32(2**30)))
            sel = li == idx
            zf = jnp.float32(0.0)
            wx1 = jnp.sum(jnp.where(sel, x1, zf))
            wy1 = jnp.sum(jnp.where(sel, y1, zf))
            wx2 = jnp.sum(jnp.where(sel, x2, zf))
            wy2 = jnp.sum(jnp.where(sel, y2, zf))
            wcls = jnp.sum(jnp.where(sel, jf, zf))
            woff = wcls * _MAX_WH
            wbx1 = wx1 + woff
            wby1 = wy1 + woff
            wbx2 = wx2 + woff
            wby2 = wy2 + woff
            xx1 = jnp.maximum(wbx1, bx1)
            yy1 = jnp.maximum(wby1, by1)
            xx2 = jnp.minimum(wbx2, bx2)
            yy2 = jnp.minimum(wby2, by2)
            inter = (jnp.maximum(xx2 - xx1, 0.0) * jnp.maximum(yy2 - yy1, 0.0))
            a1 = (wbx2 - wbx1) * (wby2 - wby1)
            iou = inter / (a1 + a2 - inter + jnp.float32(1e-7))
            s2 = jnp.where(iou > _IOU_THRES, _NEG_INF, s)
            s2 = jnp.where(sel, _NEG_INF, s2)
            new_ss.append(jnp.where(ok, s2, s))
            okf = jnp.where(ok, jnp.float32(1.0), jnp.float32(0.0))
            row = jnp.concatenate(
                [v.reshape(1, 1) * okf for v in (wx1, wy1, wx2, wy2, m, wcls)],
                axis=1)
            out_ref[b, pl.ds(t, 1), :] = row
        return tuple(new_ss)

    init = tuple(per_img[b][0] for b in range(_NIMG))
    lax.fori_loop(0, _MAX_DET, step, init)


def kernel(x):
    pred = x[0]
    pad = jnp.zeros((_NIMG, _NPAD - _N, pred.shape[-1]), pred.dtype)
    p = jnp.concatenate([pred, pad], axis=1)
    pt = p.reshape(_NIMG, _ROWS, _COLS, pred.shape[-1]).transpose(0, 3, 1, 2)
    out = pl.pallas_call(
        _nms_body,
        out_shape=jax.ShapeDtypeStruct((_NIMG, _MAX_DET, 6), jnp.float32),
    )(pt)
    return out


# speedup vs baseline: 24.1580x; 24.1580x over previous
import jax
import jax.numpy as jnp
from jax import lax
from jax.experimental import pallas as pl

_CONF_THRES = 0.25
_IOU_THRES = 0.45
_MAX_DET = 1000
_MAX_WH = 4096.0

_N = 20000
_NPAD = 20480
_ROWS = 160
_COLS = 128
_NCLS = 80
_NIMG = 4

_NEG_INF = float("-inf")


def _nms_body(p_ref, out_ref):
    li = (lax.broadcasted_iota(jnp.int32, (_ROWS, _COLS), 0) * _COLS
          + lax.broadcasted_iota(jnp.int32, (_ROWS, _COLS), 1))

    per_img = []
    for b in range(_NIMG):
        cx = p_ref[b, 0]
        cy = p_ref[b, 1]
        w = p_ref[b, 2]
        h = p_ref[b, 3]
        obj = p_ref[b, 4]
        x1 = cx - w / 2
        y1 = cy - h / 2
        x2 = cx + w / 2
        y2 = cy + h / 2
        best = p_ref[b, 5] * obj
        jbest = jnp.zeros((_ROWS, _COLS), jnp.int32)
        for c in range(1, _NCLS):
            v = p_ref[b, 5 + c] * obj
            take = v > best
            jbest = jnp.where(take, c, jbest)
            best = jnp.maximum(best, v)
        conf = best
        valid = (obj > _CONF_THRES) & (conf > _CONF_THRES)
        scores = jnp.where(valid, conf, _NEG_INF)
        offs = jbest.astype(jnp.float32) * _MAX_WH
        bx1 = x1 + offs
        by1 = y1 + offs
        bx2 = x2 + offs
        by2 = y2 + offs
        a2 = (bx2 - bx1) * (by2 - by1)
        jf = jbest.astype(jnp.float32)
        per_img.append((scores, x1, y1, x2, y2, bx1, by1, bx2, by2, a2, jf))

    def step(t, ss):
        new_ss = []
        for b in range(_NIMG):
            (_, x1, y1, x2, y2, bx1, by1, bx2, by2, a2, jf) = per_img[b]
            s = ss[b]
            m = jnp.max(s)
            ok = m > 0.0
            eq = s == m
            idx = jnp.min(jnp.where(eq, li, 2**30))
            sel = li == idx
            wx1 = jnp.sum(jnp.where(sel, x1, 0.0))
            wy1 = jnp.sum(jnp.where(sel, y1, 0.0))
            wx2 = jnp.sum(jnp.where(sel, x2, 0.0))
            wy2 = jnp.sum(jnp.where(sel, y2, 0.0))
            wcls = jnp.sum(jnp.where(sel, jf, 0.0))
            woff = wcls * _MAX_WH
            wbx1 = wx1 + woff
            wby1 = wy1 + woff
            wbx2 = wx2 + woff
            wby2 = wy2 + woff
            xx1 = jnp.maximum(wbx1, bx1)
            yy1 = jnp.maximum(wby1, by1)
            xx2 = jnp.minimum(wbx2, bx2)
            yy2 = jnp.minimum(wby2, by2)
            inter = (jnp.maximum(xx2 - xx1, 0.0) * jnp.maximum(yy2 - yy1, 0.0))
            a1 = (wbx2 - wbx1) * (wby2 - wby1)
            iou = inter / (a1 + a2 - inter + 1e-7)
            s2 = jnp.where(iou > _IOU_THRES, _NEG_INF, s)
            s2 = jnp.where(sel, _NEG_INF, s2)
            new_ss.append(jnp.where(ok, s2, s))
            row = jnp.concatenate(
                [jnp.where(ok, v, 0.0).reshape(1, 1)
                 for v in (wx1, wy1, wx2, wy2, m, wcls)],
                axis=1)
            out_ref[b, pl.ds(t, 1), :] = row
        return tuple(new_ss)

    init = tuple(per_img[b][0] for b in range(_NIMG))
    lax.fori_loop(0, _MAX_DET, step, init)


def kernel(x):
    pred = x[0]
    pad = jnp.zeros((_NIMG, _NPAD - _N, pred.shape[-1]), pred.dtype)
    p = jnp.concatenate([pred, pad], axis=1)
    pt = p.reshape(_NIMG, _ROWS, _COLS, pred.shape[-1]).transpose(0, 3, 1, 2)
    out = pl.pallas_call(
        _nms_body,
        out_shape=jax.ShapeDtypeStruct((_NIMG, _MAX_DET, 6), jnp.float32),
    )(pt)
    return out
